# Initial kernel scaffold; baseline (speedup 1.0000x reference)
import jax
import jax.numpy as jnp
from jax.experimental import pallas as pl


def kernel(source_points, source_fea, target_points, target_fea, W1, b1, W2, b2, W3, b3, W4, b4, W5, b5, W6, b6):
    raise NotImplementedError("write your pallas kernel here")


# speedup vs baseline: 11.7268x; 11.7268x over previous
import functools

import jax
import jax.numpy as jnp
from jax.experimental import pallas as pl
from jax.experimental.pallas import tpu as pltpu
from jax.experimental.pallas import tpu_sc as plsc

N = 10000
NPAD = 10240
KN = 20
QBLK = 256
BIGF = 3.0e38


def _bf16_rtne(x):
    b = jax.lax.bitcast_convert_type(x, jnp.uint32)
    rnd = jnp.uint32(0x7FFF) + ((b >> 16) & jnp.uint32(1))
    return jax.lax.bitcast_convert_type((b + rnd) & jnp.uint32(0xFFFF0000),
                                        jnp.float32)


def _knn_body(qref, ktref, oref, dscr):
    q0 = qref[:, 0:1]
    q1 = qref[:, 1:2]
    q2 = qref[:, 2:3]
    sqq = qref[:, 3:4]
    k0 = ktref[0:1, :]
    k1 = ktref[1:2, :]
    k2 = ktref[2:3, :]
    sqk = ktref[3:4, :]
    inner = q0 * k0 + (q1 * k1 + q2 * k2)
    dscr[...] = (sqq - 2.0 * inner) + sqk

    lane = jax.lax.broadcasted_iota(jnp.int32, (QBLK, NPAD), 1)
    ocol = jax.lax.broadcasted_iota(jnp.int32, (QBLK, 128), 1)

    def it(j, acc):
        d = dscr[...]
        m = jnp.min(d, axis=1, keepdims=True)
        eq = d == m
        am = jnp.min(jnp.where(eq, lane, NPAD), axis=1, keepdims=True)
        dscr[...] = jnp.where(eq & (lane == am), BIGF, d)
        return jnp.where(ocol == j, am, acc)

    oref[...] = jax.lax.fori_loop(0, KN, it, jnp.zeros((QBLK, 128), jnp.int32))


@jax.jit
def _knn(points):
    sq = jnp.sum(points * points, axis=-1)
    pb = _bf16_rtne(points)
    pad_q = jnp.zeros((NPAD - N, 8), jnp.float32)
    qs = jnp.concatenate([pb, sq[:, None],
                          jnp.zeros((N, 4), jnp.float32)], axis=1)
    qs = jnp.concatenate([qs, pad_q], axis=0)
    kpad = jnp.full((NPAD - N, 3), 1.0e5, jnp.float32)
    kb = jnp.concatenate([pb, kpad], axis=0)
    sqk = jnp.concatenate([sq, jnp.full((NPAD - N,), 3.0e10, jnp.float32)])
    kt = jnp.concatenate([kb.T, sqk[None, :],
                          jnp.zeros((4, NPAD), jnp.float32)], axis=0)
    out = pl.pallas_call(
        _knn_body,
        grid=(NPAD // QBLK,),
        in_specs=[
            pl.BlockSpec((QBLK, 8), lambda i: (i, 0)),
            pl.BlockSpec((8, NPAD), lambda i: (0, 0)),
        ],
        out_specs=pl.BlockSpec((QBLK, 128), lambda i: (i, 0)),
        out_shape=jax.ShapeDtypeStruct((NPAD, 128), jnp.int32),
        scratch_shapes=[pltpu.VMEM((QBLK, NPAD), jnp.float32)],
    )(qs, kt)
    return out[:N, :KN]


def _mm_body(xref, wref, bref, oref):
    acc = jnp.dot(xref[...], wref[...], preferred_element_type=jnp.float32)
    oref[...] = acc + bref[...]


def _matmul(x, w, b, blk=2048):
    npts, cin = x.shape
    cout = w.shape[1]
    return pl.pallas_call(
        _mm_body,
        grid=(npts // blk,),
        in_specs=[
            pl.BlockSpec((blk, cin), lambda i: (i, 0)),
            pl.BlockSpec((cin, cout), lambda i: (0, 0)),
            pl.BlockSpec((1, cout), lambda i: (0, 0)),
        ],
        out_specs=pl.BlockSpec((blk, cout), lambda i: (i, 0)),
        out_shape=jax.ShapeDtypeStruct((npts, cout), jnp.float32),
    )(x, w, b[None, :])


_SC_INFO = None


def _sc_mesh():
    return plsc.VectorSubcoreMesh(core_axis_name="c", subcore_axis_name="s")


def _gather_max(A, C, idx3, cout):
    nw = 32
    rows_pw = NPAD // nw
    chunk = 32
    nchunks = rows_pw // chunk
    gsz = chunk * KN

    awidth = A.shape[1]

    @functools.partial(
        pl.kernel,
        mesh=_sc_mesh(),
        out_type=jax.ShapeDtypeStruct((NPAD, cout), jnp.float32),
        scratch_types=[
            pltpu.VMEM((KN, chunk), jnp.int32),
            pltpu.VMEM((gsz, awidth), jnp.float32),
            pltpu.VMEM((chunk, cout), jnp.float32),
            pltpu.VMEM((chunk, cout), jnp.float32),
            pltpu.SemaphoreType.DMA,
        ],
    )
    def k(a_hbm, c_hbm, idx_hbm, out_hbm, idx_v, rows_v, cv, accv, sem):
        wid = jax.lax.axis_index("s") * 2 + jax.lax.axis_index("c")
        base = wid * rows_pw
        cvec = cout // 16

        def do_chunk(ci, _):
            rbase = base + ci * chunk
            pltpu.sync_copy(idx_hbm.at[(base // chunk) + ci], idx_v)
            cps = [pltpu.async_copy(a_hbm.at[idx_v.at[j]],
                                    rows_v.at[pl.ds(j * chunk, chunk)], sem)
                   for j in range(KN)]
            pltpu.sync_copy(c_hbm.at[pl.ds(rbase, chunk)], cv)
            for cp in cps:
                cp.wait()

            def one_vec(v, _):
                r = v // cvec
                cc = (v % cvec) * 16
                acc = rows_v[r, pl.ds(cc, 16)]
                for j in range(1, KN):
                    acc = jnp.maximum(acc, rows_v[j * chunk + r, pl.ds(cc, 16)])
                h = acc + cv[r, pl.ds(cc, 16)]
                accv[r, pl.ds(cc, 16)] = jnp.where(h >= 0.0, h, 0.2 * h)
                return 0

            jax.lax.fori_loop(0, chunk * cvec, one_vec, 0)
            pltpu.sync_copy(accv, out_hbm.at[pl.ds(rbase, chunk)])
            return 0

        jax.lax.fori_loop(0, nchunks, do_chunk, 0)

    return k(A, C, idx3)


def _final_body(xref, w5ref, b5ref, w6ref, b6ref, oref):
    h = jnp.dot(xref[...], w5ref[...], preferred_element_type=jnp.float32)
    h = h + b5ref[...]
    h = jnp.where(h >= 0.0, h, 0.2 * h)
    o = jnp.dot(h, w6ref[...], preferred_element_type=jnp.float32)
    o = o + b6ref[...]
    m = jnp.mean(o, axis=1, keepdims=True)
    cdiff = o - m
    var = jnp.sum(cdiff * cdiff, axis=1, keepdims=True) * (1.0 / 15.0)
    oref[...] = cdiff / (jnp.sqrt(var) + 1e-07)


def _final(xc, W5, b5, W6, b6, blk=2048):
    return pl.pallas_call(
        _final_body,
        grid=(NPAD // blk,),
        in_specs=[
            pl.BlockSpec((blk, 256), lambda i: (i, 0)),
            pl.BlockSpec((256, 256), lambda i: (0, 0)),
            pl.BlockSpec((1, 256), lambda i: (0, 0)),
            pl.BlockSpec((256, 16), lambda i: (0, 0)),
            pl.BlockSpec((1, 16), lambda i: (0, 0)),
        ],
        out_specs=pl.BlockSpec((blk, 16), lambda i: (i, 0)),
        out_shape=jax.ShapeDtypeStruct((NPAD, 16), jnp.float32),
    )(xc, W5, b5[None, :], W6, b6[None, :])


def _conv(fea_pad, idx3, W, b):
    cin = fea_pad.shape[1]
    cout = W.shape[1]
    Wn, Wc = W[:cin], W[cin:]
    wn128 = jnp.concatenate(
        [Wn, jnp.zeros((cin, 128 - cout), jnp.float32)], axis=1)
    wcat = jnp.concatenate([wn128, Wc - Wn], axis=1)
    bcat = jnp.concatenate([jnp.zeros((128,), jnp.float32), b])
    ac = _matmul(fea_pad, wcat, bcat)
    A, C = ac[:, :128], ac[:, 128:]
    return _gather_max(A, C, idx3, cout)


def _cloud(points, fea, params):
    W1, b1, W2, b2, W3, b3, W4, b4, W5, b5, W6, b6 = params
    idx = _knn(points)
    pad_rows = jnp.broadcast_to(
        jnp.arange(N, NPAD, dtype=jnp.int32)[:, None], (NPAD - N, KN))
    idx_pad = jnp.concatenate([idx, pad_rows], axis=0)
    idx3 = jnp.transpose(idx_pad.reshape(NPAD // 32, 32, KN), (0, 2, 1))
    idx3 = jnp.asarray(idx3, jnp.int32)
    fea_pad = jnp.concatenate(
        [fea, jnp.zeros((NPAD - N, fea.shape[1]), jnp.float32)], axis=0)
    x1 = _conv(fea_pad, idx3, W1, b1)
    x2 = _conv(x1, idx3, W2, b2)
    x3 = _conv(x2, idx3, W3, b3)
    x4 = _conv(x3, idx3, W4, b4)
    xc = jnp.concatenate([x1, x2, x3, x4], axis=1)
    out = _final(xc, W5, b5, W6, b6)
    return out[:N]


def kernel(source_points, source_fea, target_points, target_fea,
           W1, b1, W2, b2, W3, b3, W4, b4, W5, b5, W6, b6):
    params = (W1, b1, W2, b2, W3, b3, W4, b4, W5, b5, W6, b6)
    sf = _cloud(source_points[0], source_fea[0], params)
    tf = _cloud(target_points[0], target_fea[0], params)
    sf = jnp.concatenate([source_points, sf[None]], axis=2)
    tf = jnp.concatenate([target_points, tf[None]], axis=2)
    return (sf, tf)


# speedup vs baseline: 13.3765x; 1.1407x over previous
import functools

import jax
import jax.numpy as jnp
from jax.experimental import pallas as pl
from jax.experimental.pallas import tpu as pltpu
from jax.experimental.pallas import tpu_sc as plsc

N = 10000
NPAD = 10240
KN = 20
QBLK = 256
BIGF = 3.0e38


def _bf16_rtne(x):
    b = jax.lax.bitcast_convert_type(x, jnp.uint32)
    rnd = jnp.uint32(0x7FFF) + ((b >> 16) & jnp.uint32(1))
    return jax.lax.bitcast_convert_type((b + rnd) & jnp.uint32(0xFFFF0000),
                                        jnp.float32)


def _knn_body(qref, ktref, oref, dscr):
    q0 = qref[:, 0:1]
    q1 = qref[:, 1:2]
    q2 = qref[:, 2:3]
    sqq = qref[:, 3:4]
    k0 = ktref[0:1, :]
    k1 = ktref[1:2, :]
    k2 = ktref[2:3, :]
    sqk = ktref[3:4, :]
    inner = q0 * k0 + (q1 * k1 + q2 * k2)
    dscr[...] = (sqq - 2.0 * inner) + sqk

    lane = jax.lax.broadcasted_iota(jnp.int32, (QBLK, NPAD), 1)
    ocol = jax.lax.broadcasted_iota(jnp.int32, (QBLK, 128), 1)

    def it(j, acc):
        d = dscr[...]
        am = jnp.argmin(d, axis=1).astype(jnp.int32)[:, None]
        dscr[...] = jnp.where(lane == am, BIGF, d)
        return jnp.where(ocol == j, am, acc)

    oref[...] = jax.lax.fori_loop(0, KN, it, jnp.zeros((QBLK, 128), jnp.int32))


@jax.jit
def _knn(points):
    sq = jnp.sum(points * points, axis=-1)
    pb = _bf16_rtne(points)
    pad_q = jnp.zeros((NPAD - N, 8), jnp.float32)
    qs = jnp.concatenate([pb, sq[:, None],
                          jnp.zeros((N, 4), jnp.float32)], axis=1)
    qs = jnp.concatenate([qs, pad_q], axis=0)
    kpad = jnp.full((NPAD - N, 3), 1.0e5, jnp.float32)
    kb = jnp.concatenate([pb, kpad], axis=0)
    sqk = jnp.concatenate([sq, jnp.full((NPAD - N,), 3.0e10, jnp.float32)])
    kt = jnp.concatenate([kb.T, sqk[None, :],
                          jnp.zeros((4, NPAD), jnp.float32)], axis=0)
    out = pl.pallas_call(
        _knn_body,
        grid=(NPAD // QBLK,),
        in_specs=[
            pl.BlockSpec((QBLK, 8), lambda i: (i, 0)),
            pl.BlockSpec((8, NPAD), lambda i: (0, 0)),
        ],
        out_specs=pl.BlockSpec((QBLK, 128), lambda i: (i, 0)),
        out_shape=jax.ShapeDtypeStruct((NPAD, 128), jnp.int32),
        scratch_shapes=[pltpu.VMEM((QBLK, NPAD), jnp.float32)],
    )(qs, kt)
    return out[:N, :KN]


def _mm_body(xref, wref, bref, oref):
    acc = jnp.dot(xref[...], wref[...], preferred_element_type=jnp.float32)
    oref[...] = acc + bref[...]


def _matmul(x, w, b, blk=2048):
    npts, cin = x.shape
    cout = w.shape[1]
    return pl.pallas_call(
        _mm_body,
        grid=(npts // blk,),
        in_specs=[
            pl.BlockSpec((blk, cin), lambda i: (i, 0)),
            pl.BlockSpec((cin, cout), lambda i: (0, 0)),
            pl.BlockSpec((1, cout), lambda i: (0, 0)),
        ],
        out_specs=pl.BlockSpec((blk, cout), lambda i: (i, 0)),
        out_shape=jax.ShapeDtypeStruct((npts, cout), jnp.float32),
    )(x, w, b[None, :])


_SC_INFO = None


def _sc_mesh():
    return plsc.VectorSubcoreMesh(core_axis_name="c", subcore_axis_name="s")


def _gather_max(A, C, idx3, cout):
    nw = 32
    rows_pw = NPAD // nw
    chunk = 32
    nchunks = rows_pw // chunk
    gsz = chunk * KN

    awidth = A.shape[1]

    @functools.partial(
        pl.kernel,
        mesh=_sc_mesh(),
        out_type=jax.ShapeDtypeStruct((NPAD, cout), jnp.float32),
        scratch_types=[
            pltpu.VMEM((KN, chunk), jnp.int32),
            pltpu.VMEM((gsz, awidth), jnp.float32),
            pltpu.VMEM((chunk, cout), jnp.float32),
            pltpu.VMEM((chunk, cout), jnp.float32),
            pltpu.SemaphoreType.DMA,
        ],
    )
    def k(a_hbm, c_hbm, idx_hbm, out_hbm, idx_v, rows_v, cv, accv, sem):
        wid = jax.lax.axis_index("s") * 2 + jax.lax.axis_index("c")
        base = wid * rows_pw
        cvec = cout // 16

        def do_chunk(ci, _):
            rbase = base + ci * chunk
            pltpu.sync_copy(idx_hbm.at[(base // chunk) + ci], idx_v)
            cps = [pltpu.async_copy(a_hbm.at[idx_v.at[j]],
                                    rows_v.at[pl.ds(j * chunk, chunk)], sem)
                   for j in range(KN)]
            pltpu.sync_copy(c_hbm.at[pl.ds(rbase, chunk)], cv)
            for cp in cps:
                cp.wait()

            def one_vec(v, _):
                r = v // cvec
                cc = (v % cvec) * 16
                acc = rows_v[r, pl.ds(cc, 16)]
                for j in range(1, KN):
                    acc = jnp.maximum(acc, rows_v[j * chunk + r, pl.ds(cc, 16)])
                h = acc + cv[r, pl.ds(cc, 16)]
                accv[r, pl.ds(cc, 16)] = jnp.where(h >= 0.0, h, 0.2 * h)
                return 0

            jax.lax.fori_loop(0, chunk * cvec, one_vec, 0)
            pltpu.sync_copy(accv, out_hbm.at[pl.ds(rbase, chunk)])
            return 0

        jax.lax.fori_loop(0, nchunks, do_chunk, 0)

    return k(A, C, idx3)


def _final_body(xref, w5ref, b5ref, w6ref, b6ref, oref):
    h = jnp.dot(xref[...], w5ref[...], preferred_element_type=jnp.float32)
    h = h + b5ref[...]
    h = jnp.where(h >= 0.0, h, 0.2 * h)
    o = jnp.dot(h, w6ref[...], preferred_element_type=jnp.float32)
    o = o + b6ref[...]
    m = jnp.mean(o, axis=1, keepdims=True)
    cdiff = o - m
    var = jnp.sum(cdiff * cdiff, axis=1, keepdims=True) * (1.0 / 15.0)
    oref[...] = cdiff / (jnp.sqrt(var) + 1e-07)


def _final(xc, W5, b5, W6, b6, blk=2048):
    return pl.pallas_call(
        _final_body,
        grid=(NPAD // blk,),
        in_specs=[
            pl.BlockSpec((blk, 256), lambda i: (i, 0)),
            pl.BlockSpec((256, 256), lambda i: (0, 0)),
            pl.BlockSpec((1, 256), lambda i: (0, 0)),
            pl.BlockSpec((256, 16), lambda i: (0, 0)),
            pl.BlockSpec((1, 16), lambda i: (0, 0)),
        ],
        out_specs=pl.BlockSpec((blk, 16), lambda i: (i, 0)),
        out_shape=jax.ShapeDtypeStruct((NPAD, 16), jnp.float32),
    )(xc, W5, b5[None, :], W6, b6[None, :])


def _conv(fea_pad, idx3, W, b):
    cin = fea_pad.shape[1]
    cout = W.shape[1]
    Wn, Wc = W[:cin], W[cin:]
    wn128 = jnp.concatenate(
        [Wn, jnp.zeros((cin, 128 - cout), jnp.float32)], axis=1)
    wcat = jnp.concatenate([wn128, Wc - Wn], axis=1)
    bcat = jnp.concatenate([jnp.zeros((128,), jnp.float32), b])
    ac = _matmul(fea_pad, wcat, bcat)
    A, C = ac[:, :128], ac[:, 128:]
    return _gather_max(A, C, idx3, cout)


def _cloud(points, fea, params):
    W1, b1, W2, b2, W3, b3, W4, b4, W5, b5, W6, b6 = params
    idx = _knn(points)
    pad_rows = jnp.broadcast_to(
        jnp.arange(N, NPAD, dtype=jnp.int32)[:, None], (NPAD - N, KN))
    idx_pad = jnp.concatenate([idx, pad_rows], axis=0)
    idx3 = jnp.transpose(idx_pad.reshape(NPAD // 32, 32, KN), (0, 2, 1))
    idx3 = jnp.asarray(idx3, jnp.int32)
    fea_pad = jnp.concatenate(
        [fea, jnp.zeros((NPAD - N, fea.shape[1]), jnp.float32)], axis=0)
    x1 = _conv(fea_pad, idx3, W1, b1)
    x2 = _conv(x1, idx3, W2, b2)
    x3 = _conv(x2, idx3, W3, b3)
    x4 = _conv(x3, idx3, W4, b4)
    xc = jnp.concatenate([x1, x2, x3, x4], axis=1)
    out = _final(xc, W5, b5, W6, b6)
    return out[:N]


def kernel(source_points, source_fea, target_points, target_fea,
           W1, b1, W2, b2, W3, b3, W4, b4, W5, b5, W6, b6):
    params = (W1, b1, W2, b2, W3, b3, W4, b4, W5, b5, W6, b6)
    sf = _cloud(source_points[0], source_fea[0], params)
    tf = _cloud(target_points[0], target_fea[0], params)
    sf = jnp.concatenate([source_points, sf[None]], axis=2)
    tf = jnp.concatenate([target_points, tf[None]], axis=2)
    return (sf, tf)


# speedup vs baseline: 13.6810x; 1.0228x over previous
import functools

import jax
import jax.numpy as jnp
from jax.experimental import pallas as pl
from jax.experimental.pallas import tpu as pltpu
from jax.experimental.pallas import tpu_sc as plsc

N = 10000
NPAD = 10240
KN = 20
QBLK = 256
BIGF = 3.0e38


def _bf16_rtne(x):
    b = jax.lax.bitcast_convert_type(x, jnp.uint32)
    rnd = jnp.uint32(0x7FFF) + ((b >> 16) & jnp.uint32(1))
    return jax.lax.bitcast_convert_type((b + rnd) & jnp.uint32(0xFFFF0000),
                                        jnp.float32)


def _knn_body(qref, ktref, oref, dscr):
    q0 = qref[:, 0:1]
    q1 = qref[:, 1:2]
    q2 = qref[:, 2:3]
    sqq = qref[:, 3:4]
    k0 = ktref[0:1, :]
    k1 = ktref[1:2, :]
    k2 = ktref[2:3, :]
    sqk = ktref[3:4, :]
    inner = q0 * k0 + (q1 * k1 + q2 * k2)
    dscr[...] = (sqq - 2.0 * inner) + sqk

    lane = jax.lax.broadcasted_iota(jnp.int32, (QBLK, NPAD), 1)
    ocol = jax.lax.broadcasted_iota(jnp.int32, (QBLK, 128), 1)

    def it(j, acc):
        d = dscr[...]
        am = jnp.argmin(d, axis=1).astype(jnp.int32)[:, None]
        dscr[...] = jnp.where(lane == am, BIGF, d)
        return jnp.where(ocol == j, am, acc)

    oref[...] = jax.lax.fori_loop(0, KN, it, jnp.zeros((QBLK, 128), jnp.int32))


@jax.jit
def _knn(points):
    sq = jnp.sum(points * points, axis=-1)
    pb = _bf16_rtne(points)
    pad_q = jnp.zeros((NPAD - N, 8), jnp.float32)
    qs = jnp.concatenate([pb, sq[:, None],
                          jnp.zeros((N, 4), jnp.float32)], axis=1)
    qs = jnp.concatenate([qs, pad_q], axis=0)
    kpad = jnp.full((NPAD - N, 3), 1.0e5, jnp.float32)
    kb = jnp.concatenate([pb, kpad], axis=0)
    sqk = jnp.concatenate([sq, jnp.full((NPAD - N,), 3.0e10, jnp.float32)])
    kt = jnp.concatenate([kb.T, sqk[None, :],
                          jnp.zeros((4, NPAD), jnp.float32)], axis=0)
    out = pl.pallas_call(
        _knn_body,
        grid=(NPAD // QBLK,),
        in_specs=[
            pl.BlockSpec((QBLK, 8), lambda i: (i, 0)),
            pl.BlockSpec((8, NPAD), lambda i: (0, 0)),
        ],
        out_specs=pl.BlockSpec((QBLK, 128), lambda i: (i, 0)),
        out_shape=jax.ShapeDtypeStruct((NPAD, 128), jnp.int32),
        scratch_shapes=[pltpu.VMEM((QBLK, NPAD), jnp.float32)],
    )(qs, kt)
    return out[:N, :KN]


def _mm_body(xref, wref, bref, oref):
    acc = jnp.dot(xref[...], wref[...], preferred_element_type=jnp.float32)
    oref[...] = acc + bref[...]


def _matmul(x, w, b, blk=2048):
    npts, cin = x.shape
    cout = w.shape[1]
    return pl.pallas_call(
        _mm_body,
        grid=(npts // blk,),
        in_specs=[
            pl.BlockSpec((blk, cin), lambda i: (i, 0)),
            pl.BlockSpec((cin, cout), lambda i: (0, 0)),
            pl.BlockSpec((1, cout), lambda i: (0, 0)),
        ],
        out_specs=pl.BlockSpec((blk, cout), lambda i: (i, 0)),
        out_shape=jax.ShapeDtypeStruct((npts, cout), jnp.float32),
    )(x, w, b[None, :])


_SC_INFO = None


def _sc_mesh():
    return plsc.VectorSubcoreMesh(core_axis_name="c", subcore_axis_name="s")


def _gather_max(A, C, idx3, cout):
    nw = 32
    rows_pw = NPAD // nw
    chunk = 32
    nchunks = rows_pw // chunk
    gsz = chunk * KN

    awidth = A.shape[1]

    @functools.partial(
        pl.kernel,
        mesh=_sc_mesh(),
        compiler_params=pltpu.CompilerParams(use_tc_tiling_on_sc=False),
        out_type=jax.ShapeDtypeStruct((NPAD, cout), jnp.float32),
        scratch_types=[
            pltpu.VMEM((KN, chunk), jnp.int32),
            pltpu.VMEM((gsz, awidth), jnp.float32),
            pltpu.VMEM((chunk, cout), jnp.float32),
            pltpu.VMEM((chunk, cout), jnp.float32),
            pltpu.SemaphoreType.DMA,
        ],
    )
    def k(a_hbm, c_hbm, idx_hbm, out_hbm, idx_v, rows_v, cv, accv, sem):
        wid = jax.lax.axis_index("s") * 2 + jax.lax.axis_index("c")
        base = wid * rows_pw
        cvec = cout // 16

        def do_chunk(ci, _):
            rbase = base + ci * chunk
            pltpu.sync_copy(idx_hbm.at[(base // chunk) + ci], idx_v)
            cps = [pltpu.async_copy(a_hbm.at[idx_v.at[j]],
                                    rows_v.at[pl.ds(j * chunk, chunk)], sem)
                   for j in range(KN)]
            pltpu.sync_copy(c_hbm.at[pl.ds(rbase, chunk)], cv)
            for cp in cps:
                cp.wait()

            def one_vec(v, _):
                r = v // cvec
                cc = (v % cvec) * 16
                acc = rows_v[r, pl.ds(cc, 16)]
                for j in range(1, KN):
                    acc = jnp.maximum(acc, rows_v[j * chunk + r, pl.ds(cc, 16)])
                h = acc + cv[r, pl.ds(cc, 16)]
                accv[r, pl.ds(cc, 16)] = jnp.where(h >= 0.0, h, 0.2 * h)
                return 0

            jax.lax.fori_loop(0, chunk * cvec, one_vec, 0)
            pltpu.sync_copy(accv, out_hbm.at[pl.ds(rbase, chunk)])
            return 0

        jax.lax.fori_loop(0, nchunks, do_chunk, 0)

    return k(A, C, idx3)


def _final_body(xref, w5ref, b5ref, w6ref, b6ref, oref):
    h = jnp.dot(xref[...], w5ref[...], preferred_element_type=jnp.float32)
    h = h + b5ref[...]
    h = jnp.where(h >= 0.0, h, 0.2 * h)
    o = jnp.dot(h, w6ref[...], preferred_element_type=jnp.float32)
    o = o + b6ref[...]
    m = jnp.mean(o, axis=1, keepdims=True)
    cdiff = o - m
    var = jnp.sum(cdiff * cdiff, axis=1, keepdims=True) * (1.0 / 15.0)
    oref[...] = cdiff / (jnp.sqrt(var) + 1e-07)


def _final(xc, W5, b5, W6, b6, blk=2048):
    return pl.pallas_call(
        _final_body,
        grid=(NPAD // blk,),
        in_specs=[
            pl.BlockSpec((blk, 256), lambda i: (i, 0)),
            pl.BlockSpec((256, 256), lambda i: (0, 0)),
            pl.BlockSpec((1, 256), lambda i: (0, 0)),
            pl.BlockSpec((256, 16), lambda i: (0, 0)),
            pl.BlockSpec((1, 16), lambda i: (0, 0)),
        ],
        out_specs=pl.BlockSpec((blk, 16), lambda i: (i, 0)),
        out_shape=jax.ShapeDtypeStruct((NPAD, 16), jnp.float32),
    )(xc, W5, b5[None, :], W6, b6[None, :])


def _conv(fea_pad, idx3, W, b):
    cin = fea_pad.shape[1]
    cout = W.shape[1]
    Wn, Wc = W[:cin], W[cin:]
    wcat = jnp.concatenate([Wn, Wc - Wn], axis=1)
    bcat = jnp.concatenate([jnp.zeros((cout,), jnp.float32), b])
    ac = _matmul(fea_pad, wcat, bcat)
    A, C = ac[:, :cout], ac[:, cout:]
    return _gather_max(A, C, idx3, cout)


def _cloud(points, fea, params):
    W1, b1, W2, b2, W3, b3, W4, b4, W5, b5, W6, b6 = params
    idx = _knn(points)
    pad_rows = jnp.broadcast_to(
        jnp.arange(N, NPAD, dtype=jnp.int32)[:, None], (NPAD - N, KN))
    idx_pad = jnp.concatenate([idx, pad_rows], axis=0)
    idx3 = jnp.transpose(idx_pad.reshape(NPAD // 32, 32, KN), (0, 2, 1))
    idx3 = jnp.asarray(idx3, jnp.int32)
    fea_pad = jnp.concatenate(
        [fea, jnp.zeros((NPAD - N, fea.shape[1]), jnp.float32)], axis=0)
    x1 = _conv(fea_pad, idx3, W1, b1)
    x2 = _conv(x1, idx3, W2, b2)
    x3 = _conv(x2, idx3, W3, b3)
    x4 = _conv(x3, idx3, W4, b4)
    xc = jnp.concatenate([x1, x2, x3, x4], axis=1)
    out = _final(xc, W5, b5, W6, b6)
    return out[:N]


def kernel(source_points, source_fea, target_points, target_fea,
           W1, b1, W2, b2, W3, b3, W4, b4, W5, b5, W6, b6):
    params = (W1, b1, W2, b2, W3, b3, W4, b4, W5, b5, W6, b6)
    sf = _cloud(source_points[0], source_fea[0], params)
    tf = _cloud(target_points[0], target_fea[0], params)
    sf = jnp.concatenate([source_points, sf[None]], axis=2)
    tf = jnp.concatenate([target_points, tf[None]], axis=2)
    return (sf, tf)


# speedup vs baseline: 15.5444x; 1.1362x over previous
import functools

import jax
import jax.numpy as jnp
from jax.experimental import pallas as pl
from jax.experimental.pallas import tpu as pltpu
from jax.experimental.pallas import tpu_sc as plsc

N = 10000
NPAD = 10240
KN = 20
QBLK = 256
BIGF = 3.0e38


def _bf16_rtne(x):
    b = jax.lax.bitcast_convert_type(x, jnp.uint32)
    rnd = jnp.uint32(0x7FFF) + ((b >> 16) & jnp.uint32(1))
    return jax.lax.bitcast_convert_type((b + rnd) & jnp.uint32(0xFFFF0000),
                                        jnp.float32)


def _knn_body(qref, ktref, oref, dscr):
    q0 = qref[:, 0:1]
    q1 = qref[:, 1:2]
    q2 = qref[:, 2:3]
    sqq = qref[:, 3:4]
    k0 = ktref[0:1, :]
    k1 = ktref[1:2, :]
    k2 = ktref[2:3, :]
    sqk = ktref[3:4, :]
    inner = q0 * k0 + (q1 * k1 + q2 * k2)
    dscr[...] = (sqq - 2.0 * inner) + sqk

    lane = jax.lax.broadcasted_iota(jnp.int32, (QBLK, NPAD), 1)
    ocol = jax.lax.broadcasted_iota(jnp.int32, (QBLK, 128), 1)

    def it(j, acc):
        d = dscr[...]
        am1 = jnp.argmin(d, axis=1).astype(jnp.int32)[:, None]
        d2 = jnp.where(lane == am1, BIGF, d)
        am2 = jnp.argmin(d2, axis=1).astype(jnp.int32)[:, None]
        dscr[...] = jnp.where(lane == am2, BIGF, d2)
        acc = jnp.where(ocol == 2 * j, am1, acc)
        return jnp.where(ocol == 2 * j + 1, am2, acc)

    oref[...] = jax.lax.fori_loop(0, KN // 2, it,
                                  jnp.zeros((QBLK, 128), jnp.int32))


@jax.jit
def _knn(points):
    sq = jnp.sum(points * points, axis=-1)
    pb = _bf16_rtne(points)
    pad_q = jnp.zeros((NPAD - N, 8), jnp.float32)
    qs = jnp.concatenate([pb, sq[:, None],
                          jnp.zeros((N, 4), jnp.float32)], axis=1)
    qs = jnp.concatenate([qs, pad_q], axis=0)
    kpad = jnp.full((NPAD - N, 3), 1.0e5, jnp.float32)
    kb = jnp.concatenate([pb, kpad], axis=0)
    sqk = jnp.concatenate([sq, jnp.full((NPAD - N,), 3.0e10, jnp.float32)])
    kt = jnp.concatenate([kb.T, sqk[None, :],
                          jnp.zeros((4, NPAD), jnp.float32)], axis=0)
    out = pl.pallas_call(
        _knn_body,
        grid=(NPAD // QBLK,),
        in_specs=[
            pl.BlockSpec((QBLK, 8), lambda i: (i, 0)),
            pl.BlockSpec((8, NPAD), lambda i: (0, 0)),
        ],
        out_specs=pl.BlockSpec((QBLK, 128), lambda i: (i, 0)),
        out_shape=jax.ShapeDtypeStruct((NPAD, 128), jnp.int32),
        scratch_shapes=[pltpu.VMEM((QBLK, NPAD), jnp.float32)],
        compiler_params=pltpu.CompilerParams(
            vmem_limit_bytes=63 * 1024 * 1024),
    )(qs, kt)
    return out[:N, :KN]


def _mm_body(xref, wref, bref, oref):
    acc = jnp.dot(xref[...], wref[...], preferred_element_type=jnp.float32)
    oref[...] = acc + bref[...]


def _matmul(x, w, b, blk=2048):
    npts, cin = x.shape
    cout = w.shape[1]
    return pl.pallas_call(
        _mm_body,
        grid=(npts // blk,),
        in_specs=[
            pl.BlockSpec((blk, cin), lambda i: (i, 0)),
            pl.BlockSpec((cin, cout), lambda i: (0, 0)),
            pl.BlockSpec((1, cout), lambda i: (0, 0)),
        ],
        out_specs=pl.BlockSpec((blk, cout), lambda i: (i, 0)),
        out_shape=jax.ShapeDtypeStruct((npts, cout), jnp.float32),
    )(x, w, b[None, :])


_SC_INFO = None


def _sc_mesh():
    return plsc.VectorSubcoreMesh(core_axis_name="c", subcore_axis_name="s")


def _gather_max(A, C, idx3, cout):
    nw = 32
    rows_pw = NPAD // nw
    chunk = 32
    nchunks = rows_pw // chunk
    gsz = chunk * KN

    awidth = A.shape[1]

    @functools.partial(
        pl.kernel,
        mesh=_sc_mesh(),
        compiler_params=pltpu.CompilerParams(use_tc_tiling_on_sc=False),
        out_type=jax.ShapeDtypeStruct((NPAD, cout), jnp.float32),
        scratch_types=[
            pltpu.VMEM((KN, chunk), jnp.int32),
            pltpu.VMEM((gsz, awidth), jnp.float32),
            pltpu.VMEM((chunk, cout), jnp.float32),
            pltpu.VMEM((chunk, cout), jnp.float32),
            pltpu.SemaphoreType.DMA,
        ],
    )
    def k(a_hbm, c_hbm, idx_hbm, out_hbm, idx_v, rows_v, cv, accv, sem):
        wid = jax.lax.axis_index("s") * 2 + jax.lax.axis_index("c")
        base = wid * rows_pw
        cvec = cout // 16

        def do_chunk(ci, _):
            rbase = base + ci * chunk
            pltpu.sync_copy(idx_hbm.at[(base // chunk) + ci], idx_v)
            cps = [pltpu.async_copy(a_hbm.at[idx_v.at[j]],
                                    rows_v.at[pl.ds(j * chunk, chunk)], sem)
                   for j in range(KN)]
            pltpu.sync_copy(c_hbm.at[pl.ds(rbase, chunk)], cv)
            for cp in cps:
                cp.wait()

            def one_vec(v, _):
                r = v // cvec
                cc = (v % cvec) * 16
                acc = rows_v[r, pl.ds(cc, 16)]
                for j in range(1, KN):
                    acc = jnp.maximum(acc, rows_v[j * chunk + r, pl.ds(cc, 16)])
                h = acc + cv[r, pl.ds(cc, 16)]
                accv[r, pl.ds(cc, 16)] = jnp.where(h >= 0.0, h, 0.2 * h)
                return 0

            jax.lax.fori_loop(0, chunk * cvec, one_vec, 0)
            pltpu.sync_copy(accv, out_hbm.at[pl.ds(rbase, chunk)])
            return 0

        jax.lax.fori_loop(0, nchunks, do_chunk, 0)

    return k(A, C, idx3)


def _final_body(xref, w5ref, b5ref, w6ref, b6ref, oref):
    h = jnp.dot(xref[...], w5ref[...], preferred_element_type=jnp.float32)
    h = h + b5ref[...]
    h = jnp.where(h >= 0.0, h, 0.2 * h)
    o = jnp.dot(h, w6ref[...], preferred_element_type=jnp.float32)
    o = o + b6ref[...]
    m = jnp.mean(o, axis=1, keepdims=True)
    cdiff = o - m
    var = jnp.sum(cdiff * cdiff, axis=1, keepdims=True) * (1.0 / 15.0)
    oref[...] = cdiff / (jnp.sqrt(var) + 1e-07)


def _final(xc, W5, b5, W6, b6, blk=2048):
    return pl.pallas_call(
        _final_body,
        grid=(NPAD // blk,),
        in_specs=[
            pl.BlockSpec((blk, 256), lambda i: (i, 0)),
            pl.BlockSpec((256, 256), lambda i: (0, 0)),
            pl.BlockSpec((1, 256), lambda i: (0, 0)),
            pl.BlockSpec((256, 16), lambda i: (0, 0)),
            pl.BlockSpec((1, 16), lambda i: (0, 0)),
        ],
        out_specs=pl.BlockSpec((blk, 16), lambda i: (i, 0)),
        out_shape=jax.ShapeDtypeStruct((NPAD, 16), jnp.float32),
    )(xc, W5, b5[None, :], W6, b6[None, :])


def _conv(fea_pad, idx3, W, b):
    cin = fea_pad.shape[1]
    cout = W.shape[1]
    Wn, Wc = W[:cin], W[cin:]
    wcat = jnp.concatenate([Wn, Wc - Wn], axis=1)
    bcat = jnp.concatenate([jnp.zeros((cout,), jnp.float32), b])
    ac = _matmul(fea_pad, wcat, bcat)
    A, C = ac[:, :cout], ac[:, cout:]
    return _gather_max(A, C, idx3, cout)


def _cloud(points, fea, params):
    W1, b1, W2, b2, W3, b3, W4, b4, W5, b5, W6, b6 = params
    idx = _knn(points)
    pad_rows = jnp.broadcast_to(
        jnp.arange(N, NPAD, dtype=jnp.int32)[:, None], (NPAD - N, KN))
    idx_pad = jnp.concatenate([idx, pad_rows], axis=0)
    idx3 = jnp.transpose(idx_pad.reshape(NPAD // 32, 32, KN), (0, 2, 1))
    idx3 = jnp.asarray(idx3, jnp.int32)
    fea_pad = jnp.concatenate(
        [fea, jnp.zeros((NPAD - N, fea.shape[1]), jnp.float32)], axis=0)
    x1 = _conv(fea_pad, idx3, W1, b1)
    x2 = _conv(x1, idx3, W2, b2)
    x3 = _conv(x2, idx3, W3, b3)
    x4 = _conv(x3, idx3, W4, b4)
    xc = jnp.concatenate([x1, x2, x3, x4], axis=1)
    out = _final(xc, W5, b5, W6, b6)
    return out[:N]


def kernel(source_points, source_fea, target_points, target_fea,
           W1, b1, W2, b2, W3, b3, W4, b4, W5, b5, W6, b6):
    params = (W1, b1, W2, b2, W3, b3, W4, b4, W5, b5, W6, b6)
    sf = _cloud(source_points[0], source_fea[0], params)
    tf = _cloud(target_points[0], target_fea[0], params)
    sf = jnp.concatenate([source_points, sf[None]], axis=2)
    tf = jnp.concatenate([target_points, tf[None]], axis=2)
    return (sf, tf)


# speedup vs baseline: 16.2436x; 1.0450x over previous
import functools

import jax
import jax.numpy as jnp
from jax.experimental import pallas as pl
from jax.experimental.pallas import tpu as pltpu
from jax.experimental.pallas import tpu_sc as plsc

N = 10000
NPAD = 10240
KN = 20
QBLK = 128
BIGF = 3.0e38


def _bf16_rtne(x):
    b = jax.lax.bitcast_convert_type(x, jnp.uint32)
    rnd = jnp.uint32(0x7FFF) + ((b >> 16) & jnp.uint32(1))
    return jax.lax.bitcast_convert_type((b + rnd) & jnp.uint32(0xFFFF0000),
                                        jnp.float32)


def _knn_body(qref, ktref, oref, dscr):
    q0 = qref[:, 0:1]
    q1 = qref[:, 1:2]
    q2 = qref[:, 2:3]
    sqq = qref[:, 3:4]
    k0 = ktref[0:1, :]
    k1 = ktref[1:2, :]
    k2 = ktref[2:3, :]
    sqk = ktref[3:4, :]
    inner = q0 * k0 + (q1 * k1 + q2 * k2)
    dscr[...] = (sqq - 2.0 * inner) + sqk

    lane = jax.lax.broadcasted_iota(jnp.int32, (QBLK, NPAD), 1)
    ocol = jax.lax.broadcasted_iota(jnp.int32, (QBLK, 128), 1)

    nsel = 4

    def it(j, acc):
        d = dscr[...]
        for s in range(nsel):
            am = jnp.argmin(d, axis=1).astype(jnp.int32)[:, None]
            d = jnp.where(lane == am, BIGF, d)
            acc = jnp.where(ocol == nsel * j + s, am, acc)
        dscr[...] = d
        return acc

    oref[...] = jax.lax.fori_loop(0, KN // nsel, it,
                                  jnp.zeros((QBLK, 128), jnp.int32))


@jax.jit
def _knn(points):
    sq = jnp.sum(points * points, axis=-1)
    pb = _bf16_rtne(points)
    pad_q = jnp.zeros((NPAD - N, 8), jnp.float32)
    qs = jnp.concatenate([pb, sq[:, None],
                          jnp.zeros((N, 4), jnp.float32)], axis=1)
    qs = jnp.concatenate([qs, pad_q], axis=0)
    kpad = jnp.full((NPAD - N, 3), 1.0e5, jnp.float32)
    kb = jnp.concatenate([pb, kpad], axis=0)
    sqk = jnp.concatenate([sq, jnp.full((NPAD - N,), 3.0e10, jnp.float32)])
    kt = jnp.concatenate([kb.T, sqk[None, :],
                          jnp.zeros((4, NPAD), jnp.float32)], axis=0)
    out = pl.pallas_call(
        _knn_body,
        grid=(NPAD // QBLK,),
        in_specs=[
            pl.BlockSpec((QBLK, 8), lambda i: (i, 0)),
            pl.BlockSpec((8, NPAD), lambda i: (0, 0)),
        ],
        out_specs=pl.BlockSpec((QBLK, 128), lambda i: (i, 0)),
        out_shape=jax.ShapeDtypeStruct((NPAD, 128), jnp.int32),
        scratch_shapes=[pltpu.VMEM((QBLK, NPAD), jnp.float32)],
        compiler_params=pltpu.CompilerParams(
            vmem_limit_bytes=63 * 1024 * 1024),
    )(qs, kt)
    return out[:N, :KN]


def _mm_body(xref, wref, bref, oref):
    acc = jnp.dot(xref[...], wref[...], preferred_element_type=jnp.float32)
    oref[...] = acc + bref[...]


def _matmul(x, w, b, blk=2048):
    npts, cin = x.shape
    cout = w.shape[1]
    return pl.pallas_call(
        _mm_body,
        grid=(npts // blk,),
        in_specs=[
            pl.BlockSpec((blk, cin), lambda i: (i, 0)),
            pl.BlockSpec((cin, cout), lambda i: (0, 0)),
            pl.BlockSpec((1, cout), lambda i: (0, 0)),
        ],
        out_specs=pl.BlockSpec((blk, cout), lambda i: (i, 0)),
        out_shape=jax.ShapeDtypeStruct((npts, cout), jnp.float32),
    )(x, w, b[None, :])


_SC_INFO = None


def _sc_mesh():
    return plsc.VectorSubcoreMesh(core_axis_name="c", subcore_axis_name="s")


def _gather_max(A, C, idx3, cout):
    nw = 32
    rows_pw = NPAD // nw
    chunk = 32
    nchunks = rows_pw // chunk
    gsz = chunk * KN

    awidth = A.shape[1]

    @functools.partial(
        pl.kernel,
        mesh=_sc_mesh(),
        compiler_params=pltpu.CompilerParams(use_tc_tiling_on_sc=False),
        out_type=jax.ShapeDtypeStruct((NPAD, cout), jnp.float32),
        scratch_types=[
            pltpu.VMEM((KN, chunk), jnp.int32),
            pltpu.VMEM((gsz, awidth), jnp.float32),
            pltpu.VMEM((chunk, cout), jnp.float32),
            pltpu.VMEM((chunk, cout), jnp.float32),
            pltpu.SemaphoreType.DMA,
        ],
    )
    def k(a_hbm, c_hbm, idx_hbm, out_hbm, idx_v, rows_v, cv, accv, sem):
        wid = jax.lax.axis_index("s") * 2 + jax.lax.axis_index("c")
        base = wid * rows_pw
        cvec = cout // 16

        def do_chunk(ci, _):
            rbase = base + ci * chunk
            pltpu.sync_copy(idx_hbm.at[(base // chunk) + ci], idx_v)
            cps = [pltpu.async_copy(a_hbm.at[idx_v.at[j]],
                                    rows_v.at[pl.ds(j * chunk, chunk)], sem)
                   for j in range(KN)]
            pltpu.sync_copy(c_hbm.at[pl.ds(rbase, chunk)], cv)
            for cp in cps:
                cp.wait()

            def one_vec(v, _):
                r = v // cvec
                cc = (v % cvec) * 16
                acc = rows_v[r, pl.ds(cc, 16)]
                for j in range(1, KN):
                    acc = jnp.maximum(acc, rows_v[j * chunk + r, pl.ds(cc, 16)])
                h = acc + cv[r, pl.ds(cc, 16)]
                accv[r, pl.ds(cc, 16)] = jnp.where(h >= 0.0, h, 0.2 * h)
                return 0

            jax.lax.fori_loop(0, chunk * cvec, one_vec, 0)
            pltpu.sync_copy(accv, out_hbm.at[pl.ds(rbase, chunk)])
            return 0

        jax.lax.fori_loop(0, nchunks, do_chunk, 0)

    return k(A, C, idx3)


def _final_body(xref, w5ref, b5ref, w6ref, b6ref, oref):
    h = jnp.dot(xref[...], w5ref[...], preferred_element_type=jnp.float32)
    h = h + b5ref[...]
    h = jnp.where(h >= 0.0, h, 0.2 * h)
    o = jnp.dot(h, w6ref[...], preferred_element_type=jnp.float32)
    o = o + b6ref[...]
    m = jnp.mean(o, axis=1, keepdims=True)
    cdiff = o - m
    var = jnp.sum(cdiff * cdiff, axis=1, keepdims=True) * (1.0 / 15.0)
    oref[...] = cdiff / (jnp.sqrt(var) + 1e-07)


def _final(xc, W5, b5, W6, b6, blk=2048):
    return pl.pallas_call(
        _final_body,
        grid=(NPAD // blk,),
        in_specs=[
            pl.BlockSpec((blk, 256), lambda i: (i, 0)),
            pl.BlockSpec((256, 256), lambda i: (0, 0)),
            pl.BlockSpec((1, 256), lambda i: (0, 0)),
            pl.BlockSpec((256, 16), lambda i: (0, 0)),
            pl.BlockSpec((1, 16), lambda i: (0, 0)),
        ],
        out_specs=pl.BlockSpec((blk, 16), lambda i: (i, 0)),
        out_shape=jax.ShapeDtypeStruct((NPAD, 16), jnp.float32),
    )(xc, W5, b5[None, :], W6, b6[None, :])


def _conv(fea_pad, idx3, W, b):
    cin = fea_pad.shape[1]
    cout = W.shape[1]
    Wn, Wc = W[:cin], W[cin:]
    wcat = jnp.concatenate([Wn, Wc - Wn], axis=1)
    bcat = jnp.concatenate([jnp.zeros((cout,), jnp.float32), b])
    ac = _matmul(fea_pad, wcat, bcat)
    A, C = ac[:, :cout], ac[:, cout:]
    return _gather_max(A, C, idx3, cout)


def _cloud(points, fea, params):
    W1, b1, W2, b2, W3, b3, W4, b4, W5, b5, W6, b6 = params
    idx = _knn(points)
    pad_rows = jnp.broadcast_to(
        jnp.arange(N, NPAD, dtype=jnp.int32)[:, None], (NPAD - N, KN))
    idx_pad = jnp.concatenate([idx, pad_rows], axis=0)
    idx3 = jnp.transpose(idx_pad.reshape(NPAD // 32, 32, KN), (0, 2, 1))
    idx3 = jnp.asarray(idx3, jnp.int32)
    fea_pad = jnp.concatenate(
        [fea, jnp.zeros((NPAD - N, fea.shape[1]), jnp.float32)], axis=0)
    x1 = _conv(fea_pad, idx3, W1, b1)
    x2 = _conv(x1, idx3, W2, b2)
    x3 = _conv(x2, idx3, W3, b3)
    x4 = _conv(x3, idx3, W4, b4)
    xc = jnp.concatenate([x1, x2, x3, x4], axis=1)
    out = _final(xc, W5, b5, W6, b6)
    return out[:N]


def kernel(source_points, source_fea, target_points, target_fea,
           W1, b1, W2, b2, W3, b3, W4, b4, W5, b5, W6, b6):
    params = (W1, b1, W2, b2, W3, b3, W4, b4, W5, b5, W6, b6)
    sf = _cloud(source_points[0], source_fea[0], params)
    tf = _cloud(target_points[0], target_fea[0], params)
    sf = jnp.concatenate([source_points, sf[None]], axis=2)
    tf = jnp.concatenate([target_points, tf[None]], axis=2)
    return (sf, tf)
